# Initial kernel scaffold; baseline (speedup 1.0000x reference)
import jax
import jax.numpy as jnp
from jax.experimental import pallas as pl


def kernel(x_nchw, conv_w, conv_b, gamma, beta):
    raise NotImplementedError("write your pallas kernel here")


# speedup vs baseline: 1.4508x; 1.4508x over previous
import functools

import jax
import jax.numpy as jnp
from jax.experimental import pallas as pl
from jax.experimental.pallas import tpu as pltpu

_EPS = 1e-5


def _shifts(kh, kw, wp):
    return [di * wp + dj for di in range(kh) for dj in range(kw)]


def _conv_acc(x, w_ref, shifts, m):
    acc = jnp.dot(x[shifts[0]:shifts[0] + m, :], w_ref[0],
                  preferred_element_type=jnp.float32)
    for t, s in enumerate(shifts[1:], start=1):
        acc = acc + jnp.dot(x[s:s + m, :], w_ref[t],
                            preferred_element_type=jnp.float32)
    return acc


def _stats_kernel(x_ref, w_ref, mask_ref, s_ref, *, shifts, m):
    acc = _conv_acc(x_ref[0], w_ref, shifts, m)
    ym = acc * mask_ref[...]
    s1 = jnp.sum(ym, axis=0, keepdims=True)
    s2 = jnp.sum(ym * acc, axis=0, keepdims=True)
    s_ref[0] = jnp.concatenate([s1, s2], axis=0)


def _conv_bn_relu_kernel(x_ref, w_ref, st_ref, g_ref, b_ref, o_ref,
                         *, shifts, m, count):
    acc = _conv_acc(x_ref[0], w_ref, shifts, m)
    tot = jnp.sum(st_ref[...], axis=0)
    mean = tot[0] / count
    var = jnp.maximum(tot[1] / count - mean * mean, 0.0)
    scale = g_ref[0] * jax.lax.rsqrt(var + _EPS)
    shift = b_ref[0] - mean * scale
    o_ref[0] = jnp.maximum(acc * scale[None, :] + shift[None, :], 0.0)


def kernel(x_nchw, conv_w, conv_b, gamma, beta):
    del conv_b
    N, Cin, H, W = x_nchw.shape
    Cout, cin2, kh, kw = conv_w.shape
    assert cin2 == Cin
    ph = pw = 1
    Hp, Wp = H + 2 * ph, W + 2 * pw
    Ho, Wo = Hp - kh + 1, Wp - kw + 1

    Mimg = Hp * Wp
    m = (Mimg + 7) // 8 * 8
    halo = (kh - 1) * Wp + (kw - 1)
    rows = (m + halo + 7) // 8 * 8
    shifts = _shifts(kh, kw, Wp)
    count = float(N * Ho * Wo)

    xt = jnp.transpose(x_nchw, (0, 2, 3, 1)).astype(jnp.float32)
    xp = jnp.pad(xt, ((0, 0), (ph, ph), (pw, pw), (0, 0)))
    xf = xp.reshape(N, Mimg, Cin)
    xf = jnp.pad(xf, ((0, 0), (0, rows - Mimg), (0, 0)))

    w9 = jnp.transpose(conv_w, (2, 3, 1, 0)).reshape(kh * kw, Cin, Cout)
    w9 = w9.astype(jnp.float32)

    r = jnp.arange(m)
    q = r % Mimg
    valid = (r < Mimg) & ((q // Wp) < Ho) & ((q % Wp) < Wo)
    mask = valid.astype(jnp.float32).reshape(m, 1)

    cparams = pltpu.CompilerParams(
        dimension_semantics=("parallel",),
        vmem_limit_bytes=100 * 1024 * 1024,
    )

    stats = pl.pallas_call(
        functools.partial(_stats_kernel, shifts=shifts, m=m),
        grid=(N,),
        in_specs=[
            pl.BlockSpec((1, rows, Cin), lambda n: (n, 0, 0)),
            pl.BlockSpec((kh * kw, Cin, Cout), lambda n: (0, 0, 0)),
            pl.BlockSpec((m, 1), lambda n: (0, 0)),
        ],
        out_specs=pl.BlockSpec((1, 2, Cout), lambda n: (n, 0, 0)),
        out_shape=jax.ShapeDtypeStruct((N, 2, Cout), jnp.float32),
        compiler_params=cparams,
    )(xf, w9, mask)

    out_flat = pl.pallas_call(
        functools.partial(_conv_bn_relu_kernel, shifts=shifts, m=m,
                          count=count),
        grid=(N,),
        in_specs=[
            pl.BlockSpec((1, rows, Cin), lambda n: (n, 0, 0)),
            pl.BlockSpec((kh * kw, Cin, Cout), lambda n: (0, 0, 0)),
            pl.BlockSpec((N, 2, Cout), lambda n: (0, 0, 0)),
            pl.BlockSpec((1, Cout), lambda n: (0, 0)),
            pl.BlockSpec((1, Cout), lambda n: (0, 0)),
        ],
        out_specs=pl.BlockSpec((1, m, Cout), lambda n: (n, 0, 0)),
        out_shape=jax.ShapeDtypeStruct((N, m, Cout), jnp.float32),
        compiler_params=cparams,
    )(xf, w9, stats, gamma.astype(jnp.float32).reshape(1, Cout),
      beta.astype(jnp.float32).reshape(1, Cout))

    out = out_flat[:, :Mimg, :].reshape(N, Hp, Wp, Cout)[:, :Ho, :Wo, :]
    return jnp.transpose(out, (0, 3, 1, 2))


# speedup vs baseline: 2.2713x; 1.5655x over previous
import functools

import jax
import jax.numpy as jnp
from jax.experimental import pallas as pl
from jax.experimental.pallas import tpu as pltpu

_EPS = 1e-5


def _conv_acc(xt, w_ref, shifts, m):
    acc = jnp.dot(xt[shifts[0]:shifts[0] + m, :], w_ref[0],
                  preferred_element_type=jnp.float32)
    for t, s in enumerate(shifts[1:], start=1):
        acc = acc + jnp.dot(xt[s:s + m, :], w_ref[t],
                            preferred_element_type=jnp.float32)
    return acc


def _stats_kernel(x_ref, w_ref, mask_ref, s_ref, *, shifts, m):
    xt = jnp.transpose(x_ref[0], (1, 0))
    acc = _conv_acc(xt, w_ref, shifts, m)
    ym = acc * mask_ref[...]
    s1 = jnp.sum(ym, axis=0, keepdims=True)
    s2 = jnp.sum(ym * acc, axis=0, keepdims=True)
    s_ref[0] = jnp.concatenate([s1, s2], axis=0)


def _conv_bn_relu_kernel(x_ref, w_ref, st_ref, g_ref, b_ref, o_ref,
                         *, shifts, m, count):
    xt = jnp.transpose(x_ref[0], (1, 0))
    acc = _conv_acc(xt, w_ref, shifts, m)
    tot = jnp.sum(st_ref[...], axis=0)
    mean = tot[0] / count
    var = jnp.maximum(tot[1] / count - mean * mean, 0.0)
    scale = g_ref[0] * jax.lax.rsqrt(var + _EPS)
    shift = b_ref[0] - mean * scale
    res = jnp.maximum(acc * scale[None, :] + shift[None, :], 0.0)
    o_ref[0] = jnp.transpose(res, (1, 0))


def kernel(x_nchw, conv_w, conv_b, gamma, beta):
    del conv_b
    N, Cin, H, W = x_nchw.shape
    Cout, cin2, kh, kw = conv_w.shape
    assert cin2 == Cin
    ph = pw = 1
    Hp, Wp = H + 2 * ph, W + 2 * pw
    Ho, Wo = Hp - kh + 1, Wp - kw + 1

    Mimg = Hp * Wp
    m = Ho * Wp
    m = (m + 7) // 8 * 8
    halo = (kh - 1) * Wp + (kw - 1)
    rows = (m + halo + 127) // 128 * 128
    shifts = [di * Wp + dj for di in range(kh) for dj in range(kw)]
    count = float(N * Ho * Wo)

    xp = jnp.pad(x_nchw.astype(jnp.float32),
                 ((0, 0), (0, 0), (ph, ph), (pw, pw)))
    xf = xp.reshape(N, Cin, Mimg)
    xf = jnp.pad(xf, ((0, 0), (0, 0), (0, rows - Mimg)))

    w9 = jnp.transpose(conv_w, (2, 3, 1, 0)).reshape(kh * kw, Cin, Cout)
    w9 = w9.astype(jnp.float32)

    r = jnp.arange(m)
    valid = (r < Ho * Wp) & ((r % Wp) < Wo)
    mask = valid.astype(jnp.float32).reshape(m, 1)

    cparams = pltpu.CompilerParams(
        dimension_semantics=("parallel",),
        vmem_limit_bytes=100 * 1024 * 1024,
    )

    stats = pl.pallas_call(
        functools.partial(_stats_kernel, shifts=shifts, m=m),
        grid=(N,),
        in_specs=[
            pl.BlockSpec((1, Cin, rows), lambda n: (n, 0, 0)),
            pl.BlockSpec((kh * kw, Cin, Cout), lambda n: (0, 0, 0)),
            pl.BlockSpec((m, 1), lambda n: (0, 0)),
        ],
        out_specs=pl.BlockSpec((1, 2, Cout), lambda n: (n, 0, 0)),
        out_shape=jax.ShapeDtypeStruct((N, 2, Cout), jnp.float32),
        compiler_params=cparams,
    )(xf, w9, mask)

    out_flat = pl.pallas_call(
        functools.partial(_conv_bn_relu_kernel, shifts=shifts, m=m,
                          count=count),
        grid=(N,),
        in_specs=[
            pl.BlockSpec((1, Cin, rows), lambda n: (n, 0, 0)),
            pl.BlockSpec((kh * kw, Cin, Cout), lambda n: (0, 0, 0)),
            pl.BlockSpec((N, 2, Cout), lambda n: (0, 0, 0)),
            pl.BlockSpec((1, Cout), lambda n: (0, 0)),
            pl.BlockSpec((1, Cout), lambda n: (0, 0)),
        ],
        out_specs=pl.BlockSpec((1, Cout, m), lambda n: (n, 0, 0)),
        out_shape=jax.ShapeDtypeStruct((N, Cout, m), jnp.float32),
        compiler_params=cparams,
    )(xf, w9, stats, gamma.astype(jnp.float32).reshape(1, Cout),
      beta.astype(jnp.float32).reshape(1, Cout))

    out = out_flat[:, :, :Ho * Wp].reshape(N, Cout, Ho, Wp)[:, :, :, :Wo]
    return out
